# Initial kernel scaffold; baseline (speedup 1.0000x reference)
import jax
import jax.numpy as jnp
from jax.experimental import pallas as pl


def kernel(pair_embedding, explanation_path, schema_bucket_ids, hop_counts, path_source_ids, params):
    raise NotImplementedError("write your pallas kernel here")


# speedup vs baseline: 1.8726x; 1.8726x over previous
import functools

import jax
import jax.numpy as jnp
from jax import lax
from jax.experimental import pallas as pl
from jax.experimental.pallas import tpu as pltpu
from jax.experimental.pallas import tpu_sc as plsc

F32 = jnp.float32
_N, _H, _MD, _NP, _RH = 8192, 2048, 512, 64, 2048
_SCHEMA, _MAXHOP, _SRCVOC = 2048, 8, 8
_RS = 0.2
_BN = 256
_NB = _N // _BN
_BJ = 512
_SC_C = 16
_SQRT1_2 = 0.7071067811865476


def _gelu_exact(x):
    return 0.5 * x * (1.0 + lax.erf(x * _SQRT1_2))


def _layernorm(x, g, b, eps=1e-5):
    mu = jnp.mean(x, axis=-1, keepdims=True)
    xc = x - mu
    var = jnp.mean(xc * xc, axis=-1, keepdims=True)
    return xc * lax.rsqrt(var + eps) * g + b


def _fold_body(schema_ref, hop_ref, src_ref, w_s_ref, w_h_ref, w_p_ref,
               ts_ref, thp_ref):
    ts_ref[...] = jnp.dot(schema_ref[...], w_s_ref[...],
                          preferred_element_type=F32)
    thp_ref[0:16, :] = jnp.dot(hop_ref[...], w_h_ref[...],
                               preferred_element_type=F32)
    thp_ref[16:24, :] = jnp.dot(src_ref[...], w_p_ref[...],
                                preferred_element_type=F32)
    thp_ref[24:32, :] = jnp.zeros((8, _H), F32)


def _fold_tables(schema_emb, hop_pad, source_emb, w_s, w_h, w_p):
    return pl.pallas_call(
        _fold_body,
        out_shape=(jax.ShapeDtypeStruct((_SCHEMA + 1, _H), F32),
                   jax.ShapeDtypeStruct((32, _H), F32)),
    )(schema_emb, hop_pad, source_emb, w_s, w_h, w_p)


def _sc_gather(table, idx):
    info = plsc.get_sparse_core_info()
    nw = info.num_cores * info.num_subcores
    rows_per_w = _N // nw
    n_chunks = rows_per_w // _SC_C
    mesh = plsc.VectorSubcoreMesh(core_axis_name="c", subcore_axis_name="s")

    @functools.partial(
        pl.kernel, mesh=mesh,
        out_type=jax.ShapeDtypeStruct((_N, _H), F32),
        scratch_types=[pltpu.VMEM((_SC_C,), jnp.int32),
                       pltpu.VMEM((_SC_C, _H), F32),
                       pltpu.SemaphoreType.DMA],
    )
    def gather_k(table_hbm, idx_hbm, out_hbm, idx_v, rows_v, sem):
        wid = lax.axis_index("s") * info.num_cores + lax.axis_index("c")
        base = wid * rows_per_w

        def body(c, carry):
            off = base + c * _SC_C
            pltpu.sync_copy(idx_hbm.at[pl.ds(off, _SC_C)], idx_v)
            pltpu.async_copy(table_hbm.at[idx_v], rows_v, sem).wait()
            pltpu.sync_copy(rows_v, out_hbm.at[pl.ds(off, _SC_C)])
            return carry

        lax.fori_loop(0, n_chunks, body, 0)

    return gather_k(table, idx)


def _stage_ab_body(pair_ref, ep_ref, hs_ref, hid_ref, pid_ref,
                   w_pair_ref, w_ep_ref, thp_ref, b1_ref, g1_ref, be1_ref,
                   w2_ref, b2_ref, proto_ref, ctx_ref, pw_ref):
    acc = jnp.dot(pair_ref[...], w_pair_ref[...], preferred_element_type=F32)
    acc = acc + jnp.dot(ep_ref[...], w_ep_ref[...], preferred_element_type=F32)
    acc = acc + hs_ref[...]
    hid = hid_ref[0, 0, :]
    pid = pid_ref[0, 0, :]
    cols = lax.broadcasted_iota(jnp.int32, (_BN, 32), 1)
    oh = jnp.where((hid[:, None] == cols) | ((pid[:, None] + 16) == cols),
                   1.0, 0.0).astype(F32)
    acc = acc + jnp.dot(oh, thp_ref[...], preferred_element_type=F32,
                        precision=lax.Precision.HIGHEST)
    acc = acc + b1_ref[...]
    h = _layernorm(acc, g1_ref[...], be1_ref[...])
    h = _gelu_exact(h)
    logits = jnp.dot(h, w2_ref[...], preferred_element_type=F32) + b2_ref[...]
    cur = logits
    t = None
    for _ in range(4):
        t = jnp.max(cur, axis=-1, keepdims=True)
        cur = jnp.where(cur >= t, -jnp.inf, cur)
    mask = logits >= t
    m1 = jnp.max(logits, axis=-1, keepdims=True)
    e = jnp.exp(logits - m1)
    es = jnp.where(mask, e, 0.0)
    pw = es / jnp.sum(es, axis=-1, keepdims=True)
    pw_ref[...] = pw
    ctx_ref[...] = jnp.dot(pw, proto_ref[...], preferred_element_type=F32)


def _stage_ab(pair, ep, hs, hid3, pid3, w_pair, w_ep, thp, b1, g1, be1,
              w2, b2, proto):
    return pl.pallas_call(
        _stage_ab_body,
        grid=(_NB,),
        in_specs=[
            pl.BlockSpec((_BN, _H), lambda n: (n, 0)),
            pl.BlockSpec((_BN, _H), lambda n: (n, 0)),
            pl.BlockSpec((_BN, _H), lambda n: (n, 0)),
            pl.BlockSpec((1, 1, _BN), lambda n: (n, 0, 0)),
            pl.BlockSpec((1, 1, _BN), lambda n: (n, 0, 0)),
            pl.BlockSpec((_H, _H), lambda n: (0, 0)),
            pl.BlockSpec((_H, _H), lambda n: (0, 0)),
            pl.BlockSpec((32, _H), lambda n: (0, 0)),
            pl.BlockSpec((_H,), lambda n: (0,)),
            pl.BlockSpec((_H,), lambda n: (0,)),
            pl.BlockSpec((_H,), lambda n: (0,)),
            pl.BlockSpec((_H, _NP), lambda n: (0, 0)),
            pl.BlockSpec((_NP,), lambda n: (0,)),
            pl.BlockSpec((_NP, _H), lambda n: (0, 0)),
        ],
        out_specs=(pl.BlockSpec((_BN, _H), lambda n: (n, 0)),
                   pl.BlockSpec((_BN, _NP), lambda n: (n, 0))),
        out_shape=(jax.ShapeDtypeStruct((_N, _H), F32),
                   jax.ShapeDtypeStruct((_N, _NP), F32)),
    )(pair, ep, hs, hid3, pid3, w_pair, w_ep, thp, b1, g1, be1, w2, b2, proto)


def _mm1_body(ep_ref, ctx_ref, w_ref, o_ref):
    ep = ep_ref[...]
    ctx = ctx_ref[...]
    acc = jnp.dot(ep, w_ref[0:_H, :], preferred_element_type=F32)
    acc = acc + jnp.dot(ctx, w_ref[_H:2 * _H, :], preferred_element_type=F32)
    acc = acc + jnp.dot(jnp.abs(ep - ctx), w_ref[2 * _H:3 * _H, :],
                        preferred_element_type=F32)
    acc = acc + jnp.dot(ep * ctx, w_ref[3 * _H:4 * _H, :],
                        preferred_element_type=F32)
    o_ref[...] = acc


def _mm1(ep, ctx, w):
    nj = _H // _BJ
    return pl.pallas_call(
        _mm1_body,
        grid=(nj, _NB),
        in_specs=[
            pl.BlockSpec((_BN, _H), lambda j, n: (n, 0)),
            pl.BlockSpec((_BN, _H), lambda j, n: (n, 0)),
            pl.BlockSpec((4 * _H, _BJ), lambda j, n: (0, j)),
        ],
        out_specs=pl.BlockSpec((_BN, _BJ), lambda j, n: (n, j)),
        out_shape=jax.ShapeDtypeStruct((_N, _H), F32),
    )(ep, ctx, w)


def _stage_c_body(d1_ref, g1_ref, ep_ref, mu_b1_ref, mu_g_ref, mu_be_ref,
                  mu_w2_ref, mu_b2_ref, mg_b1_ref, mg_w2_ref, mg_b2_ref,
                  n_g_ref, n_be_ref, out_ref):
    d1 = d1_ref[...] + mu_b1_ref[...]
    d1 = _layernorm(d1, mu_g_ref[...], mu_be_ref[...])
    d1 = _gelu_exact(d1)
    d = jnp.dot(d1, mu_w2_ref[...], preferred_element_type=F32) + mu_b2_ref[...]
    g1 = g1_ref[...] + mg_b1_ref[...]
    g1 = _gelu_exact(g1)
    g = jnp.dot(g1, mg_w2_ref[...], preferred_element_type=F32) + mg_b2_ref[...]
    g = jax.nn.sigmoid(g)
    u = ep_ref[...] + _RS * g * d
    out_ref[...] = _layernorm(u, n_g_ref[...], n_be_ref[...])


def _stage_c(d1, g1, ep, mu_b1, mu_g, mu_be, mu_w2, mu_b2,
             mg_b1, mg_w2, mg_b2, n_g, n_be):
    vec = pl.BlockSpec((_H,), lambda n: (0,))
    blk = pl.BlockSpec((_BN, _H), lambda n: (n, 0))
    mat = pl.BlockSpec((_H, _H), lambda n: (0, 0))
    return pl.pallas_call(
        _stage_c_body,
        grid=(_NB,),
        in_specs=[blk, blk, blk, vec, vec, vec, mat, vec, vec, mat, vec,
                  vec, vec],
        out_specs=blk,
        out_shape=jax.ShapeDtypeStruct((_N, _H), F32),
    )(d1, g1, ep, mu_b1, mu_g, mu_be, mu_w2, mu_b2, mg_b1, mg_w2, mg_b2,
      n_g, n_be)


def kernel(pair_embedding, explanation_path, schema_bucket_ids, hop_counts,
           path_source_ids, params):
    p = params
    sid = jnp.clip(schema_bucket_ids, 0, _SCHEMA).astype(jnp.int32)
    hid = jnp.clip(hop_counts, 0, _MAXHOP + 1).astype(jnp.int32)
    pid = jnp.clip(path_source_ids, 0, _SRCVOC - 1).astype(jnp.int32)

    w1 = p['r_w1']
    w_pair = w1[0:_H]
    w_ep = w1[_H:2 * _H]
    w_s = w1[2 * _H:2 * _H + _MD]
    w_h = w1[2 * _H + _MD:2 * _H + 2 * _MD]
    w_p = w1[2 * _H + 2 * _MD:]
    hop_pad = jnp.pad(p['hop_emb'], ((0, 16 - (_MAXHOP + 2)), (0, 0)))

    t_s, t_hp = _fold_tables(p['schema_emb'], hop_pad, p['source_emb'],
                             w_s, w_h, w_p)
    hs = _sc_gather(t_s, sid)

    hid3 = hid.reshape(_NB, 1, _BN)
    pid3 = pid.reshape(_NB, 1, _BN)
    ctx, pw = _stage_ab(pair_embedding, explanation_path, hs, hid3, pid3,
                        w_pair, w_ep, t_hp, p['r_b1'], p['r_g'], p['r_be'],
                        p['r_w2'], p['r_b2'], p['proto'])

    d1 = _mm1(explanation_path, ctx, p['mu_w1'])
    g1 = _mm1(explanation_path, ctx, p['mg_w1'])

    updated = _stage_c(d1, g1, explanation_path, p['mu_b1'], p['mu_g'],
                       p['mu_be'], p['mu_w2'], p['mu_b2'], p['mg_b1'],
                       p['mg_w2'], p['mg_b2'], p['n_g'], p['n_be'])
    return (updated, ctx, pw)
